# Initial kernel scaffold; baseline (speedup 1.0000x reference)
import jax
import jax.numpy as jnp
from jax.experimental import pallas as pl


def kernel(x, W1f, W1d, g1, b1, W4f, W4d, g4, b4, W5f, W5d, g5, b5, W6f, W6d, g6, b6):
    raise NotImplementedError("write your pallas kernel here")


# speedup vs baseline: 1.3756x; 1.3756x over previous
import functools

import jax
import jax.numpy as jnp
from jax.experimental import pallas as pl
from jax.experimental.pallas import tpu as pltpu

_EPS = 1e-6
_K = 16
_BIG = 3.0e38


def _blockdiag3(A):
    At = A.T
    Z = jnp.zeros_like(At)
    return jnp.concatenate([
        jnp.concatenate([At, Z, Z], axis=1),
        jnp.concatenate([Z, At, Z], axis=1),
        jnp.concatenate([Z, Z, At], axis=1)], axis=0)


def _knn_stats_kernel(nb, N, F, Co, xq_ref, xallT_ref, xall_ref, wbf_ref,
                      zwf_ref, idx_ref, stats_ref):
    xq = xq_ref[0]
    xallT = xallT_ref[0]
    sq_c = jnp.sum(xallT * xallT, axis=0, keepdims=True)
    sq_q = jnp.sum(xq * xq, axis=1, keepdims=True)
    inner = jnp.dot(xq, xallT, preferred_element_type=jnp.float32, precision=jax.lax.Precision.HIGHEST)
    dist = sq_c - 2.0 * inner + sq_q
    iota = jax.lax.broadcasted_iota(jnp.int32, (nb, N), 1)
    cols = []
    d = dist
    for _ in range(_K):
        m = jnp.min(d, axis=1, keepdims=True)
        am = jnp.min(jnp.where(d == m, iota, N), axis=1, keepdims=True)
        cols.append(am)
        d = jnp.where(iota == am, _BIG, d)
    idxb = jnp.concatenate(cols, axis=1)
    idx_ref[0] = idxb

    xall = xall_ref[0]
    zf = jnp.dot(xq, zwf_ref[...], preferred_element_type=jnp.float32, precision=jax.lax.Precision.HIGHEST)
    acc_n = jnp.zeros((1, Co), jnp.float32)
    acc_n2 = jnp.zeros((1, Co), jnp.float32)
    for t in range(_K):
        col = idxb[:, t:t + 1]
        oh = (iota == col).astype(jnp.float32)
        nbr = jnp.dot(oh, xall, preferred_element_type=jnp.float32, precision=jax.lax.Precision.HIGHEST)
        pf = jnp.dot(nbr, wbf_ref[...], preferred_element_type=jnp.float32, precision=jax.lax.Precision.HIGHEST) + zf
        s = pf * pf
        n2 = s[:, :Co] + s[:, Co:2 * Co] + s[:, 2 * Co:]
        norm = jnp.sqrt(n2) + _EPS
        acc_n = acc_n + jnp.sum(norm, axis=0, keepdims=True)
        acc_n2 = acc_n2 + jnp.sum(norm * norm, axis=0, keepdims=True)
    stats_ref[0] = jnp.concatenate([acc_n, acc_n2], axis=1)


def _apply_kernel(nb, N, F, Co, xq_ref, xall_ref, idx_ref, wbf_ref, wbd_ref,
                  zwf_ref, zwd_ref, bn_ref, out_ref):
    xq = xq_ref[0]
    xall = xall_ref[0]
    idxb = idx_ref[0]
    iota = jax.lax.broadcasted_iota(jnp.int32, (nb, N), 1)
    zf = jnp.dot(xq, zwf_ref[...], preferred_element_type=jnp.float32, precision=jax.lax.Precision.HIGHEST)
    zd = jnp.dot(xq, zwd_ref[...], preferred_element_type=jnp.float32, precision=jax.lax.Precision.HIGHEST)
    bn_scale = bn_ref[0:1, :]
    bn_bias = bn_ref[1:2, :]
    acc = jnp.zeros((nb, 3 * Co), jnp.float32)
    for t in range(_K):
        col = idxb[:, t:t + 1]
        oh = (iota == col).astype(jnp.float32)
        nbr = jnp.dot(oh, xall, preferred_element_type=jnp.float32, precision=jax.lax.Precision.HIGHEST)
        pf = jnp.dot(nbr, wbf_ref[...], preferred_element_type=jnp.float32, precision=jax.lax.Precision.HIGHEST) + zf
        pd = jnp.dot(nbr, wbd_ref[...], preferred_element_type=jnp.float32, precision=jax.lax.Precision.HIGHEST) + zd
        sf = pf * pf
        n2 = sf[:, :Co] + sf[:, Co:2 * Co] + sf[:, 2 * Co:]
        norm = jnp.sqrt(n2) + _EPS
        scal = (bn_scale * norm + bn_bias) / norm
        s3 = jnp.concatenate([scal, scal, scal], axis=1)
        p = pf * s3
        sd = p * pd
        dot3 = sd[:, :Co] + sd[:, Co:2 * Co] + sd[:, 2 * Co:]
        sq = pd * pd
        dsq = sq[:, :Co] + sq[:, Co:2 * Co] + sq[:, 2 * Co:]
        coef = jnp.where(dot3 >= 0, 0.0, dot3 / (dsq + _EPS))
        c3 = jnp.concatenate([coef, coef, coef], axis=1)
        acc = acc + (0.2 * p + 0.8 * (p - c3 * pd))
    out_ref[0] = acc * (1.0 / _K)


def _edge_stage(xfeat, Wf, Wd, gamma, beta, C, Co, nb):
    B, N, F = xfeat.shape
    nblk = N // nb
    wbf = _blockdiag3(Wf[:, :C])
    zwf = _blockdiag3(Wf[:, C:] - Wf[:, :C])
    wbd = _blockdiag3(Wd[:, :C])
    zwd = _blockdiag3(Wd[:, C:] - Wd[:, :C])
    xallT = xfeat.transpose(0, 2, 1)

    full2 = lambda s: pl.BlockSpec(s, lambda b, j: (0, 0))
    idx_out, stats = pl.pallas_call(
        functools.partial(_knn_stats_kernel, nb, N, F, Co),
        grid=(B, nblk),
        in_specs=[
            pl.BlockSpec((1, nb, F), lambda b, j: (b, j, 0)),
            pl.BlockSpec((1, F, N), lambda b, j: (b, 0, 0)),
            pl.BlockSpec((1, N, F), lambda b, j: (b, 0, 0)),
            full2(wbf.shape),
            full2(zwf.shape),
        ],
        out_specs=[
            pl.BlockSpec((1, nb, _K), lambda b, j: (b, j, 0)),
            pl.BlockSpec((1, 1, 2 * Co), lambda b, j: (b * nblk + j, 0, 0)),
        ],
        out_shape=[
            jax.ShapeDtypeStruct((B, N, _K), jnp.int32),
            jax.ShapeDtypeStruct((B * nblk, 1, 2 * Co), jnp.float32),
        ],
    )(xfeat, xallT, xfeat, wbf, zwf)

    cnt = float(B * N * _K)
    s = jnp.sum(stats.reshape(B * nblk, 2 * Co), axis=0)
    mean = s[:Co] / cnt
    var = s[Co:] / cnt - mean * mean
    bn_scale = gamma / jnp.sqrt(var + 1e-5)
    bn_bias = beta - mean * bn_scale
    bn = jnp.zeros((8, Co), jnp.float32).at[0].set(bn_scale).at[1].set(bn_bias)

    out = pl.pallas_call(
        functools.partial(_apply_kernel, nb, N, F, Co),
        grid=(B, nblk),
        in_specs=[
            pl.BlockSpec((1, nb, F), lambda b, j: (b, j, 0)),
            pl.BlockSpec((1, N, F), lambda b, j: (b, 0, 0)),
            pl.BlockSpec((1, nb, _K), lambda b, j: (b, j, 0)),
            full2(wbf.shape),
            full2(wbd.shape),
            full2(zwf.shape),
            full2(zwd.shape),
            full2(bn.shape),
        ],
        out_specs=pl.BlockSpec((1, nb, 3 * Co), lambda b, j: (b, j, 0)),
        out_shape=jax.ShapeDtypeStruct((B, N, 3 * Co), jnp.float32),
    )(xfeat, xfeat, idx_out, wbf, wbd, zwf, zwd, bn)
    return out


def _fps_kernel(B, N, S, coor_ref, out_ref):
    X = coor_ref[0]
    Y = coor_ref[1]
    Z = coor_ref[2]
    iN = jax.lax.broadcasted_iota(jnp.int32, (B, N), 1)
    iS = jax.lax.broadcasted_iota(jnp.int32, (B, S), 1)

    def body(i, st):
        dists, far, idxs = st
        idxs = jnp.where(iS == i, far, idxs)
        sel = iN == far
        cx = jnp.sum(jnp.where(sel, X, 0.0), axis=1, keepdims=True)
        cy = jnp.sum(jnp.where(sel, Y, 0.0), axis=1, keepdims=True)
        cz = jnp.sum(jnp.where(sel, Z, 0.0), axis=1, keepdims=True)
        dx = X - cx
        dy = Y - cy
        dz = Z - cz
        d = dx * dx + dy * dy + dz * dz
        dists = jnp.minimum(dists, d)
        m = jnp.max(dists, axis=1, keepdims=True)
        far = jnp.min(jnp.where(dists == m, iN, N), axis=1, keepdims=True)
        return (dists, far, idxs)

    st0 = (jnp.maximum(X * 0.0, 1e10),
           (X[:, :1] * 0.0).astype(jnp.int32),
           (X[:, :S] * 0.0).astype(jnp.int32))
    _, _, idxs = jax.lax.fori_loop(0, S, body, st0)
    out_ref[...] = idxs


def _fps(coor, S):
    B, N, _ = coor.shape
    cT = coor.transpose(2, 0, 1)
    return pl.pallas_call(
        functools.partial(_fps_kernel, B, N, S),
        out_shape=jax.ShapeDtypeStruct((B, S), jnp.int32),
    )(cT)


def _row_gather_kernel(S, N, comb_ref, idx_ref, out_ref):
    idxc = idx_ref[0]
    oh = (jax.lax.broadcasted_iota(jnp.int32, (S, N), 1) == idxc)
    out_ref[0] = jnp.dot(oh.astype(jnp.float32), comb_ref[0],
                         preferred_element_type=jnp.float32, precision=jax.lax.Precision.HIGHEST)


def _row_gather(comb, idx):
    B, N, Fc = comb.shape
    S = idx.shape[1]
    return pl.pallas_call(
        functools.partial(_row_gather_kernel, S, N),
        grid=(B,),
        in_specs=[
            pl.BlockSpec((1, N, Fc), lambda b: (b, 0, 0)),
            pl.BlockSpec((1, S, 1), lambda b: (b, 0, 0)),
        ],
        out_specs=pl.BlockSpec((1, S, Fc), lambda b: (b, 0, 0)),
        out_shape=jax.ShapeDtypeStruct((B, S, Fc), jnp.float32),
    )(comb, idx[:, :, None])


def kernel(x, W1f, W1d, g1, b1, W4f, W4d, g4, b4, W5f, W5d, g5, b5,
           W6f, W6d, g6, b6):
    B, _, N = x.shape
    xf0 = x.transpose(0, 2, 1)
    f1 = _edge_stage(xf0, W1f, W1d, g1, b1, 1, 32, 512)
    idx1 = _fps(xf0, 512)
    comb = jnp.concatenate([xf0, f1], axis=2)
    comb_q = _row_gather(comb, idx1)
    coor_q, fq = comb_q[:, :, :3], comb_q[:, :, 3:]
    f2 = _edge_stage(fq, W4f, W4d, g4, b4, 32, 64, 512)
    f3 = _edge_stage(f2, W5f, W5d, g5, b5, 64, 64, 512)
    idx2 = _fps(coor_q, 128)
    comb2 = jnp.concatenate([coor_q, f3], axis=2)
    comb2_q = _row_gather(comb2, idx2)
    coor2, fq2 = comb2_q[:, :, :3], comb2_q[:, :, 3:]
    f4 = _edge_stage(fq2, W6f, W6d, g6, b6, 64, 128, 128)
    return (coor2.transpose(0, 2, 1),
            f4.reshape(B, 128, 3, 128).transpose(0, 3, 2, 1))


# speedup vs baseline: 4.5347x; 3.2966x over previous
import functools

import jax
import jax.numpy as jnp
from jax.experimental import pallas as pl
from jax.experimental.pallas import tpu as pltpu
from jax.experimental.pallas import tpu_sc as plsc

_EPS = 1e-6
_K = 16
_BIG = 3.0e38
_HI = jax.lax.Precision.HIGHEST


def _blockdiag3(A):
    At = A.T
    Z = jnp.zeros_like(At)
    return jnp.concatenate([
        jnp.concatenate([At, Z, Z], axis=1),
        jnp.concatenate([Z, At, Z], axis=1),
        jnp.concatenate([Z, Z, At], axis=1)], axis=0)


def _knn_kernel(nb, N, F, xq_ref, xallT_ref, gidx_ref):
    b = pl.program_id(0)
    xq = xq_ref[0]
    xallT = xallT_ref[0]
    sq_c = jnp.sum(xallT * xallT, axis=0, keepdims=True)
    sq_q = jnp.sum(xq * xq, axis=1, keepdims=True)
    inner = jnp.dot(xq, xallT, preferred_element_type=jnp.float32, precision=_HI)
    dist = sq_c - 2.0 * inner + sq_q
    iota = jax.lax.broadcasted_iota(jnp.int32, (nb, N), 1)
    cols = []
    d = dist
    for _ in range(_K):
        m = jnp.min(d, axis=1, keepdims=True)
        am = jnp.min(jnp.where(d == m, iota, N), axis=1, keepdims=True)
        cols.append(am + b * N)
        d = jnp.where(iota == am, _BIG, d)
    gidx_ref[0] = jnp.concatenate(cols, axis=0)


def _sc_gather(table, gidx2, W):
    R = gidx2.shape[1]
    F = table.shape[1]
    mesh = plsc.VectorSubcoreMesh(core_axis_name="c", subcore_axis_name="s")

    @pl.kernel(out_type=jax.ShapeDtypeStruct((R, F), table.dtype), mesh=mesh)
    def k(tab_hbm, i_hbm, o_hbm):
        def body(i_vmem, o_vmem):
            pltpu.sync_copy(tab_hbm.at[i_vmem.at[0]], o_vmem)
        pltpu.emit_pipeline(
            body,
            grid=(R // W,),
            in_specs=[pl.BlockSpec((1, W), lambda i: (0, i))],
            out_specs=[pl.BlockSpec((W, F), lambda i: (i, 0))],
            core_axis_name=("c", "s"),
            dimension_semantics=(pltpu.PARALLEL,),
        )(i_hbm, o_hbm)

    return k(table, gidx2)


def _stats_kernel(nb, Co, xq_ref, nbr_ref, wbf_ref, zwf_ref, stats_ref):
    xq = xq_ref[0]
    zf = jnp.dot(xq, zwf_ref[...], preferred_element_type=jnp.float32, precision=_HI)
    acc_n = jnp.zeros((1, Co), jnp.float32)
    acc_n2 = jnp.zeros((1, Co), jnp.float32)
    for t in range(_K):
        nbr = nbr_ref[0, t * nb:(t + 1) * nb, :]
        pf = jnp.dot(nbr, wbf_ref[...], preferred_element_type=jnp.float32, precision=_HI) + zf
        s = pf * pf
        n2 = s[:, :Co] + s[:, Co:2 * Co] + s[:, 2 * Co:]
        norm = jnp.sqrt(n2) + _EPS
        acc_n = acc_n + jnp.sum(norm, axis=0, keepdims=True)
        acc_n2 = acc_n2 + jnp.sum(norm * norm, axis=0, keepdims=True)
    stats_ref[0] = jnp.concatenate([acc_n, acc_n2], axis=1)


def _apply_kernel(nb, Co, xq_ref, nbr_ref, wbf_ref, wbd_ref,
                  zwf_ref, zwd_ref, bn_ref, out_ref):
    xq = xq_ref[0]
    zf = jnp.dot(xq, zwf_ref[...], preferred_element_type=jnp.float32, precision=_HI)
    zd = jnp.dot(xq, zwd_ref[...], preferred_element_type=jnp.float32, precision=_HI)
    bn_scale = bn_ref[0:1, :]
    bn_bias = bn_ref[1:2, :]
    acc = jnp.zeros((nb, 3 * Co), jnp.float32)
    for t in range(_K):
        nbr = nbr_ref[0, t * nb:(t + 1) * nb, :]
        pf = jnp.dot(nbr, wbf_ref[...], preferred_element_type=jnp.float32, precision=_HI) + zf
        pd = jnp.dot(nbr, wbd_ref[...], preferred_element_type=jnp.float32, precision=_HI) + zd
        sf = pf * pf
        n2 = sf[:, :Co] + sf[:, Co:2 * Co] + sf[:, 2 * Co:]
        norm = jnp.sqrt(n2) + _EPS
        scal = (bn_scale * norm + bn_bias) / norm
        s3 = jnp.concatenate([scal, scal, scal], axis=1)
        p = pf * s3
        sd = p * pd
        dot3 = sd[:, :Co] + sd[:, Co:2 * Co] + sd[:, 2 * Co:]
        sq = pd * pd
        dsq = sq[:, :Co] + sq[:, Co:2 * Co] + sq[:, 2 * Co:]
        coef = jnp.where(dot3 >= 0, 0.0, dot3 / (dsq + _EPS))
        c3 = jnp.concatenate([coef, coef, coef], axis=1)
        acc = acc + (0.2 * p + 0.8 * (p - c3 * pd))
    out_ref[0] = acc * (1.0 / _K)


def _edge_stage(xfeat, Wf, Wd, gamma, beta, C, Co, nb):
    B, N, F = xfeat.shape
    nblk = N // nb
    Fp = ((F + 127) // 128) * 128
    pad = Fp - F
    wbf = _blockdiag3(Wf[:, :C])
    zwf = _blockdiag3(Wf[:, C:] - Wf[:, :C])
    wbd = _blockdiag3(Wd[:, :C])
    zwd = _blockdiag3(Wd[:, C:] - Wd[:, :C])
    wbfp = jnp.pad(wbf, ((0, pad), (0, 0)))
    wbdp = jnp.pad(wbd, ((0, pad), (0, 0)))
    xallT = xfeat.transpose(0, 2, 1)

    full2 = lambda s: pl.BlockSpec(s, lambda b, j: (0, 0))
    gidx = pl.pallas_call(
        functools.partial(_knn_kernel, nb, N, F),
        grid=(B, nblk),
        in_specs=[
            pl.BlockSpec((1, nb, F), lambda b, j: (b, j, 0)),
            pl.BlockSpec((1, F, N), lambda b, j: (b, 0, 0)),
        ],
        out_specs=pl.BlockSpec((1, _K * nb, 1), lambda b, j: (b * nblk + j, 0, 0)),
        out_shape=jax.ShapeDtypeStruct((B * nblk, _K * nb, 1), jnp.int32),
    )(xfeat, xallT)

    tab = xfeat.reshape(B * N, F)
    if pad:
        tab = jnp.pad(tab, ((0, 0), (0, pad)))
    R = B * N * _K
    nbr_flat = _sc_gather(tab, gidx.reshape(1, R), 128)
    nbr_rows = nbr_flat.reshape(B * nblk, _K * nb, Fp)

    stats = pl.pallas_call(
        functools.partial(_stats_kernel, nb, Co),
        grid=(B, nblk),
        in_specs=[
            pl.BlockSpec((1, nb, F), lambda b, j: (b, j, 0)),
            pl.BlockSpec((1, _K * nb, Fp), lambda b, j: (b * nblk + j, 0, 0)),
            full2(wbfp.shape),
            full2(zwf.shape),
        ],
        out_specs=pl.BlockSpec((1, 1, 2 * Co), lambda b, j: (b * nblk + j, 0, 0)),
        out_shape=jax.ShapeDtypeStruct((B * nblk, 1, 2 * Co), jnp.float32),
    )(xfeat, nbr_rows, wbfp, zwf)

    cnt = float(B * N * _K)
    s = jnp.sum(stats.reshape(B * nblk, 2 * Co), axis=0)
    mean = s[:Co] / cnt
    var = s[Co:] / cnt - mean * mean
    bn_scale = gamma / jnp.sqrt(var + 1e-5)
    bn_bias = beta - mean * bn_scale
    bn = jnp.zeros((8, Co), jnp.float32).at[0].set(bn_scale).at[1].set(bn_bias)

    out = pl.pallas_call(
        functools.partial(_apply_kernel, nb, Co),
        grid=(B, nblk),
        in_specs=[
            pl.BlockSpec((1, nb, F), lambda b, j: (b, j, 0)),
            pl.BlockSpec((1, _K * nb, Fp), lambda b, j: (b * nblk + j, 0, 0)),
            full2(wbfp.shape),
            full2(wbdp.shape),
            full2(zwf.shape),
            full2(zwd.shape),
            full2(bn.shape),
        ],
        out_specs=pl.BlockSpec((1, nb, 3 * Co), lambda b, j: (b, j, 0)),
        out_shape=jax.ShapeDtypeStruct((B, N, 3 * Co), jnp.float32),
    )(xfeat, nbr_rows, wbfp, wbdp, zwf, zwd, bn)
    return out


def _fps_kernel(B, N, S, coor_ref, out_ref):
    X = coor_ref[0]
    Y = coor_ref[1]
    Z = coor_ref[2]
    iN = jax.lax.broadcasted_iota(jnp.int32, (B, N), 1)
    iS = jax.lax.broadcasted_iota(jnp.int32, (B, S), 1)

    def body(i, st):
        dists, far, idxs = st
        idxs = jnp.where(iS == i, far, idxs)
        sel = iN == far
        cx = jnp.sum(jnp.where(sel, X, 0.0), axis=1, keepdims=True)
        cy = jnp.sum(jnp.where(sel, Y, 0.0), axis=1, keepdims=True)
        cz = jnp.sum(jnp.where(sel, Z, 0.0), axis=1, keepdims=True)
        dx = X - cx
        dy = Y - cy
        dz = Z - cz
        d = dx * dx + dy * dy + dz * dz
        dists = jnp.minimum(dists, d)
        m = jnp.max(dists, axis=1, keepdims=True)
        far = jnp.min(jnp.where(dists == m, iN, N), axis=1, keepdims=True)
        return (dists, far, idxs)

    st0 = (jnp.maximum(X * 0.0, 1e10),
           (X[:, :1] * 0.0).astype(jnp.int32),
           (X[:, :S] * 0.0).astype(jnp.int32))
    _, _, idxs = jax.lax.fori_loop(0, S, body, st0)
    out_ref[...] = idxs


def _fps(coor, S):
    B, N, _ = coor.shape
    cT = coor.transpose(2, 0, 1)
    return pl.pallas_call(
        functools.partial(_fps_kernel, B, N, S),
        out_shape=jax.ShapeDtypeStruct((B, S), jnp.int32),
    )(cT)


def _row_gather_kernel(S, N, comb_ref, idx_ref, out_ref):
    idxc = idx_ref[0]
    oh = (jax.lax.broadcasted_iota(jnp.int32, (S, N), 1) == idxc)
    out_ref[0] = jnp.dot(oh.astype(jnp.float32), comb_ref[0],
                         preferred_element_type=jnp.float32, precision=_HI)


def _row_gather(comb, idx):
    B, N, Fc = comb.shape
    S = idx.shape[1]
    return pl.pallas_call(
        functools.partial(_row_gather_kernel, S, N),
        grid=(B,),
        in_specs=[
            pl.BlockSpec((1, N, Fc), lambda b: (b, 0, 0)),
            pl.BlockSpec((1, S, 1), lambda b: (b, 0, 0)),
        ],
        out_specs=pl.BlockSpec((1, S, Fc), lambda b: (b, 0, 0)),
        out_shape=jax.ShapeDtypeStruct((B, S, Fc), jnp.float32),
    )(comb, idx[:, :, None])


def kernel(x, W1f, W1d, g1, b1, W4f, W4d, g4, b4, W5f, W5d, g5, b5,
           W6f, W6d, g6, b6):
    B, _, N = x.shape
    xf0 = x.transpose(0, 2, 1)
    f1 = _edge_stage(xf0, W1f, W1d, g1, b1, 1, 32, 512)
    idx1 = _fps(xf0, 512)
    comb = jnp.concatenate([xf0, f1], axis=2)
    comb_q = _row_gather(comb, idx1)
    coor_q, fq = comb_q[:, :, :3], comb_q[:, :, 3:]
    f2 = _edge_stage(fq, W4f, W4d, g4, b4, 32, 64, 512)
    f3 = _edge_stage(f2, W5f, W5d, g5, b5, 64, 64, 512)
    idx2 = _fps(coor_q, 128)
    comb2 = jnp.concatenate([coor_q, f3], axis=2)
    comb2_q = _row_gather(comb2, idx2)
    coor2, fq2 = comb2_q[:, :, :3], comb2_q[:, :, 3:]
    f4 = _edge_stage(fq2, W6f, W6d, g6, b6, 64, 128, 128)
    return (coor2.transpose(0, 2, 1),
            f4.reshape(B, 128, 3, 128).transpose(0, 3, 2, 1))
